# Initial kernel scaffold; baseline (speedup 1.0000x reference)
import jax
import jax.numpy as jnp
from jax.experimental import pallas as pl


def kernel(hidden_states, W1, b1, W2, b2, router_W, router_b, expert_bias):
    raise NotImplementedError("write your pallas kernel here")


# speedup vs baseline: 1.6697x; 1.6697x over previous
import functools

import jax
import jax.numpy as jnp
from jax import lax
from jax.experimental import pallas as pl
from jax.experimental.pallas import tpu as pltpu
from jax.experimental.pallas import tpu_sc as plsc

E = 8
TOP_K = 2
D = 768
F = 3072
T = 2048
BLK = 256
NB = T * TOP_K // BLK + E
P = NB * BLK
CH = 128
NCH = T // CH

_PREC = lax.Precision.HIGHEST


def _router_kernel(x_ref, rw_ref, rb_ref, eb_ref,
                   d0_ref, d1_ref, w0_ref, w1_ref, be_ref, exc_ref):
    x = x_ref[...]
    logits = jnp.dot(x, rw_ref[...], preferred_element_type=jnp.float32,
                     precision=lax.Precision.DEFAULT)
    logits = logits + rb_ref[...] + eb_ref[...]
    m = jnp.max(logits, axis=1, keepdims=True)
    ex = jnp.exp(logits - m)
    probs = ex / jnp.sum(ex, axis=1, keepdims=True)

    ei = lax.broadcasted_iota(jnp.int32, (T, E), 1)
    m1 = jnp.max(probs, axis=1, keepdims=True)
    i1 = jnp.min(jnp.where(probs == m1, ei, E), axis=1, keepdims=True)
    pmask = jnp.where(ei == i1, -1.0, probs)
    m2 = jnp.max(pmask, axis=1, keepdims=True)
    i2 = jnp.min(jnp.where(pmask == m2, ei, E), axis=1, keepdims=True)
    s = m1 + m2 + 1e-9
    w0_ref[...] = jnp.broadcast_to(m1 / s, (T, 16))
    w1_ref[...] = jnp.broadcast_to(m2 / s, (T, 16))

    oh1 = (ei == i1).astype(jnp.float32)
    oh2 = (ei == i2).astype(jnp.float32)
    occ = oh1 + oh2

    li = lax.broadcasted_iota(jnp.int32, (CH, CH), 0)
    lj = lax.broadcasted_iota(jnp.int32, (CH, CH), 1)
    lex = (lj < li).astype(jnp.float32)
    ones_row = jnp.ones((1, CH), jnp.float32)
    off = jnp.zeros((1, E), jnp.float32)
    for c in range(NCH):
        oc = occ[c * CH:(c + 1) * CH, :]
        exc_ref[c * CH:(c + 1) * CH, :] = off + jnp.dot(
            lex, oc, preferred_element_type=jnp.float32, precision=_PREC)
        off = off + jnp.dot(ones_row, oc,
                            preferred_element_type=jnp.float32,
                            precision=_PREC)
    counts = off

    pc = jnp.ceil(counts * (1.0 / BLK)) * BLK
    gi = lax.broadcasted_iota(jnp.int32, (E, E), 0)
    gj = lax.broadcasted_iota(jnp.int32, (E, E), 1)
    gmat = (gi < gj).astype(jnp.float32)
    gs = jnp.dot(pc, gmat, preferred_element_type=jnp.float32,
                 precision=_PREC)

    exc = exc_ref[...]
    slot = gs + exc
    d0_ref[...] = jnp.sum(oh1 * slot, axis=1, keepdims=True).astype(jnp.int32)
    d1_ref[...] = jnp.sum(oh2 * slot, axis=1, keepdims=True).astype(jnp.int32)

    brow = lax.broadcasted_iota(jnp.int32, (32, E), 0).astype(jnp.float32) * BLK
    ecol = lax.broadcasted_iota(jnp.int32, (32, E), 1)
    act = jnp.logical_and(brow >= gs, brow < gs + pc)
    be = jnp.sum(jnp.where(act, ecol + 1, 0), axis=1, keepdims=True) - 1
    be_ref[...] = be


def _route(x2d, router_W, router_b, expert_bias):
    out_shapes = (
        jax.ShapeDtypeStruct((T, 1), jnp.int32),
        jax.ShapeDtypeStruct((T, 1), jnp.int32),
        jax.ShapeDtypeStruct((T, 16), jnp.float32),
        jax.ShapeDtypeStruct((T, 16), jnp.float32),
        jax.ShapeDtypeStruct((32, 1), jnp.int32),
    )
    return pl.pallas_call(
        _router_kernel,
        out_shape=out_shapes,
        scratch_shapes=[pltpu.VMEM((T, E), jnp.float32)],
    )(x2d, router_W, router_b.reshape(1, E), expert_bias.reshape(1, E))


def _ffn_kernel(be_sref, x_ref, w1_ref, b1_ref, w2_ref, b2_ref, y_ref):
    b = pl.program_id(0)

    @pl.when(be_sref[b] >= 0)
    def _active():
        h = jnp.dot(x_ref[...], w1_ref[0], preferred_element_type=jnp.float32,
                    precision=_PREC) + b1_ref[0]
        h = h * 0.5 * (1.0 + lax.erf(h * (2.0 ** -0.5)))
        y_ref[...] = jnp.dot(h, w2_ref[0], preferred_element_type=jnp.float32,
                             precision=_PREC) + b2_ref[0]

    @pl.when(be_sref[b] < 0)
    def _inactive():
        y_ref[...] = jnp.zeros_like(y_ref)


def _ffn(x_sorted, block_expert, W1, b1, W2, b2):
    def wmap(b, be):
        return (jnp.maximum(be[b], 0), 0, 0)

    grid_spec = pltpu.PrefetchScalarGridSpec(
        num_scalar_prefetch=1,
        grid=(NB,),
        in_specs=[
            pl.BlockSpec((BLK, D), lambda b, be: (b, 0)),
            pl.BlockSpec((1, D, F), wmap),
            pl.BlockSpec((1, 1, F), wmap),
            pl.BlockSpec((1, F, D), wmap),
            pl.BlockSpec((1, 1, D), wmap),
        ],
        out_specs=pl.BlockSpec((BLK, D), lambda b, be: (b, 0)),
    )
    return pl.pallas_call(
        _ffn_kernel,
        grid_spec=grid_spec,
        out_shape=jax.ShapeDtypeStruct((P, D), jnp.float32),
    )(block_expert, x_sorted, W1, b1.reshape(E, 1, F), W2,
      b2.reshape(E, 1, D))


_NC = 2
_NS = 16
_NW = _NC * _NS
_TPW = T // _NW


_CTPW = 32


@functools.cache
def _sc_kernels():
    mesh = plsc.VectorSubcoreMesh(core_axis_name="c", subcore_axis_name="s")

    @functools.partial(
        pl.kernel,
        mesh=mesh,
        out_type=jax.ShapeDtypeStruct((P, D), jnp.float32),
        scratch_types=[
            pltpu.VMEM((_TPW,), jnp.int32),
            pltpu.VMEM((_TPW,), jnp.int32),
            pltpu.VMEM((_TPW, D), jnp.float32),
            pltpu.SemaphoreType.DMA,
        ],
    )
    def dispatch(x_hbm, d0_hbm, d1_hbm, xs_hbm, idx0_v, idx1_v, rows_v, sem):
        wid = lax.axis_index("s") * _NC + lax.axis_index("c")
        base = wid * _TPW
        pltpu.sync_copy(d0_hbm.at[pl.ds(base, _TPW)], idx0_v)
        pltpu.sync_copy(d1_hbm.at[pl.ds(base, _TPW)], idx1_v)
        pltpu.sync_copy(x_hbm.at[pl.ds(base, _TPW)], rows_v)
        pltpu.async_copy(rows_v, xs_hbm.at[idx0_v], sem).wait()
        pltpu.async_copy(rows_v, xs_hbm.at[idx1_v], sem).wait()

    @functools.partial(
        pl.kernel,
        mesh=mesh,
        out_type=jax.ShapeDtypeStruct((T, D), jnp.float32),
        scratch_types=[
            pltpu.VMEM((_CTPW,), jnp.int32),
            pltpu.VMEM((_CTPW,), jnp.int32),
            pltpu.VMEM((_CTPW, 16), jnp.float32),
            pltpu.VMEM((_CTPW, 16), jnp.float32),
            pltpu.VMEM((_CTPW, D), jnp.float32),
            pltpu.VMEM((_CTPW, D), jnp.float32),
            pltpu.VMEM((_CTPW, D), jnp.float32),
            pltpu.SemaphoreType.DMA,
        ],
    )
    def combine(y_hbm, d0_hbm, d1_hbm, w0_hbm, w1_hbm, out_hbm,
                idx0_v, idx1_v, w0_v, w1_v, buf0, buf1, outb, sem):
        wid = lax.axis_index("s") * _NC + lax.axis_index("c")
        for c in range(_TPW // _CTPW):
            base = wid * _TPW + c * _CTPW
            pltpu.sync_copy(d0_hbm.at[pl.ds(base, _CTPW)], idx0_v)
            pltpu.sync_copy(d1_hbm.at[pl.ds(base, _CTPW)], idx1_v)
            pltpu.sync_copy(w0_hbm.at[pl.ds(base, _CTPW)], w0_v)
            pltpu.sync_copy(w1_hbm.at[pl.ds(base, _CTPW)], w1_v)
            g0 = pltpu.async_copy(y_hbm.at[idx0_v], buf0, sem)
            g1 = pltpu.async_copy(y_hbm.at[idx1_v], buf1, sem)
            g0.wait()
            g1.wait()

            def row_body(r, _):
                a = w0_v[r]
                b = w1_v[r]

                def col_body(j, _):
                    sl = pl.ds(pl.multiple_of(j * 16, 16), 16)
                    outb[r, sl] = a * buf0[r, sl] + b * buf1[r, sl]
                    return 0

                return lax.fori_loop(0, D // 16, col_body, 0)

            lax.fori_loop(0, _CTPW, row_body, 0)
            pltpu.sync_copy(outb, out_hbm.at[pl.ds(base, _CTPW)])

    return dispatch, combine


def kernel(hidden_states, W1, b1, W2, b2, router_W, router_b, expert_bias):
    bsz, seq, dim = hidden_states.shape
    x2d = hidden_states.reshape(T, D)
    d0, d1, w0, w1, be = _route(x2d, router_W, router_b, expert_bias)
    d0 = d0.reshape(T)
    d1 = d1.reshape(T)
    be = be.reshape(32)[:NB]
    dispatch, combine = _sc_kernels()
    x_sorted = dispatch(x2d, d0, d1)
    y_sorted = _ffn(x_sorted, be, W1, b1, W2, b2)
    out = combine(y_sorted, d0, d1, w0, w1)
    return out.reshape(bsz, seq, dim)


# speedup vs baseline: 4.0572x; 2.4300x over previous
import functools

import jax
import jax.numpy as jnp
from jax import lax
from jax.experimental import pallas as pl
from jax.experimental.pallas import tpu as pltpu
from jax.experimental.pallas import tpu_sc as plsc

E = 8
TOP_K = 2
D = 768
F = 3072
T = 2048
BLK = 256
NB = T * TOP_K // BLK + E
P = NB * BLK
CH = 128
NCH = T // CH

_PREC = lax.Precision.HIGHEST


def _router_kernel(x_ref, rw_ref, rb_ref, eb_ref,
                   d0_ref, d1_ref, w0_ref, w1_ref, be_ref, exc_ref):
    x = x_ref[...]
    logits = jnp.dot(x, rw_ref[...], preferred_element_type=jnp.float32,
                     precision=lax.Precision.DEFAULT)
    logits = logits + rb_ref[...] + eb_ref[...]
    m = jnp.max(logits, axis=1, keepdims=True)
    ex = jnp.exp(logits - m)
    probs = ex / jnp.sum(ex, axis=1, keepdims=True)

    ei = lax.broadcasted_iota(jnp.int32, (T, E), 1)
    m1 = jnp.max(probs, axis=1, keepdims=True)
    i1 = jnp.min(jnp.where(probs == m1, ei, E), axis=1, keepdims=True)
    pmask = jnp.where(ei == i1, -1.0, probs)
    m2 = jnp.max(pmask, axis=1, keepdims=True)
    i2 = jnp.min(jnp.where(pmask == m2, ei, E), axis=1, keepdims=True)
    s = m1 + m2 + 1e-9
    w0_ref[...] = jnp.broadcast_to(m1 / s, (T, 16))
    w1_ref[...] = jnp.broadcast_to(m2 / s, (T, 16))

    oh1 = (ei == i1).astype(jnp.float32)
    oh2 = (ei == i2).astype(jnp.float32)
    occ = oh1 + oh2

    li = lax.broadcasted_iota(jnp.int32, (CH, CH), 0)
    lj = lax.broadcasted_iota(jnp.int32, (CH, CH), 1)
    lex = (lj < li).astype(jnp.float32)
    ones_row = jnp.ones((1, CH), jnp.float32)
    off = jnp.zeros((1, E), jnp.float32)
    for c in range(NCH):
        oc = occ[c * CH:(c + 1) * CH, :]
        exc_ref[c * CH:(c + 1) * CH, :] = off + jnp.dot(
            lex, oc, preferred_element_type=jnp.float32, precision=_PREC)
        off = off + jnp.dot(ones_row, oc,
                            preferred_element_type=jnp.float32,
                            precision=_PREC)
    counts = off

    pc = jnp.ceil(counts * (1.0 / BLK)) * BLK
    gi = lax.broadcasted_iota(jnp.int32, (E, E), 0)
    gj = lax.broadcasted_iota(jnp.int32, (E, E), 1)
    gmat = (gi < gj).astype(jnp.float32)
    gs = jnp.dot(pc, gmat, preferred_element_type=jnp.float32,
                 precision=_PREC)

    exc = exc_ref[...]
    slot = gs + exc
    d0_ref[...] = jnp.sum(oh1 * slot, axis=1, keepdims=True).astype(jnp.int32)
    d1_ref[...] = jnp.sum(oh2 * slot, axis=1, keepdims=True).astype(jnp.int32)

    brow = lax.broadcasted_iota(jnp.int32, (32, E), 0).astype(jnp.float32) * BLK
    ecol = lax.broadcasted_iota(jnp.int32, (32, E), 1)
    act = jnp.logical_and(brow >= gs, brow < gs + pc)
    be = jnp.sum(jnp.where(act, ecol + 1, 0), axis=1, keepdims=True) - 1
    be_ref[...] = be


def _route(x2d, router_W, router_b, expert_bias):
    out_shapes = (
        jax.ShapeDtypeStruct((T, 1), jnp.int32),
        jax.ShapeDtypeStruct((T, 1), jnp.int32),
        jax.ShapeDtypeStruct((T, 16), jnp.float32),
        jax.ShapeDtypeStruct((T, 16), jnp.float32),
        jax.ShapeDtypeStruct((32, 1), jnp.int32),
    )
    return pl.pallas_call(
        _router_kernel,
        out_shape=out_shapes,
        scratch_shapes=[pltpu.VMEM((T, E), jnp.float32)],
    )(x2d, router_W, router_b.reshape(1, E), expert_bias.reshape(1, E))


def _ffn_kernel(be_sref, x_ref, w1_ref, b1_ref, w2_ref, b2_ref, y_ref):
    b = pl.program_id(0)

    @pl.when(be_sref[b] >= 0)
    def _active():
        h = jnp.dot(x_ref[...], w1_ref[0], preferred_element_type=jnp.float32,
                    precision=lax.Precision.DEFAULT) + b1_ref[0]
        h = h * 0.5 * (1.0 + lax.erf(h * (2.0 ** -0.5)))
        y_ref[...] = jnp.dot(h, w2_ref[0], preferred_element_type=jnp.float32,
                             precision=lax.Precision.DEFAULT) + b2_ref[0]

    @pl.when(be_sref[b] < 0)
    def _inactive():
        y_ref[...] = jnp.zeros_like(y_ref)


def _ffn(x_sorted, block_expert, W1, b1, W2, b2):
    def wmap(b, be):
        return (jnp.maximum(be[b], 0), 0, 0)

    grid_spec = pltpu.PrefetchScalarGridSpec(
        num_scalar_prefetch=1,
        grid=(NB,),
        in_specs=[
            pl.BlockSpec((BLK, D), lambda b, be: (b, 0)),
            pl.BlockSpec((1, D, F), wmap),
            pl.BlockSpec((1, 1, F), wmap),
            pl.BlockSpec((1, F, D), wmap),
            pl.BlockSpec((1, 1, D), wmap),
        ],
        out_specs=pl.BlockSpec((BLK, D), lambda b, be: (b, 0)),
    )
    return pl.pallas_call(
        _ffn_kernel,
        grid_spec=grid_spec,
        out_shape=jax.ShapeDtypeStruct((P, D), jnp.float32),
    )(block_expert, x_sorted, W1, b1.reshape(E, 1, F), W2,
      b2.reshape(E, 1, D))


_NC = 2
_NS = 16
_NW = _NC * _NS
_TPW = T // _NW


_CTPW = 32


@functools.cache
def _sc_kernels():
    mesh = plsc.VectorSubcoreMesh(core_axis_name="c", subcore_axis_name="s")

    @functools.partial(
        pl.kernel,
        mesh=mesh,
        out_type=jax.ShapeDtypeStruct((P, D), jnp.float32),
        scratch_types=[
            pltpu.VMEM((_TPW,), jnp.int32),
            pltpu.VMEM((_TPW,), jnp.int32),
            pltpu.VMEM((_TPW, D), jnp.float32),
            pltpu.SemaphoreType.DMA,
        ],
    )
    def dispatch(x_hbm, d0_hbm, d1_hbm, xs_hbm, idx0_v, idx1_v, rows_v, sem):
        wid = lax.axis_index("s") * _NC + lax.axis_index("c")
        base = wid * _TPW
        pltpu.sync_copy(d0_hbm.at[pl.ds(base, _TPW)], idx0_v)
        pltpu.sync_copy(d1_hbm.at[pl.ds(base, _TPW)], idx1_v)
        pltpu.sync_copy(x_hbm.at[pl.ds(base, _TPW)], rows_v)
        pltpu.async_copy(rows_v, xs_hbm.at[idx0_v], sem).wait()
        pltpu.async_copy(rows_v, xs_hbm.at[idx1_v], sem).wait()

    @functools.partial(
        pl.kernel,
        mesh=mesh,
        out_type=jax.ShapeDtypeStruct((T, D), jnp.float32),
        scratch_types=[
            pltpu.VMEM((_CTPW,), jnp.int32),
            pltpu.VMEM((_CTPW,), jnp.int32),
            pltpu.VMEM((_CTPW, 16), jnp.float32),
            pltpu.VMEM((_CTPW, 16), jnp.float32),
            pltpu.VMEM((_CTPW, D), jnp.float32),
            pltpu.VMEM((_CTPW, D), jnp.float32),
            pltpu.VMEM((_CTPW, D), jnp.float32),
            pltpu.SemaphoreType.DMA,
        ],
    )
    def combine(y_hbm, d0_hbm, d1_hbm, w0_hbm, w1_hbm, out_hbm,
                idx0_v, idx1_v, w0_v, w1_v, buf0, buf1, outb, sem):
        wid = lax.axis_index("s") * _NC + lax.axis_index("c")
        for c in range(_TPW // _CTPW):
            base = wid * _TPW + c * _CTPW
            pltpu.sync_copy(d0_hbm.at[pl.ds(base, _CTPW)], idx0_v)
            pltpu.sync_copy(d1_hbm.at[pl.ds(base, _CTPW)], idx1_v)
            pltpu.sync_copy(w0_hbm.at[pl.ds(base, _CTPW)], w0_v)
            pltpu.sync_copy(w1_hbm.at[pl.ds(base, _CTPW)], w1_v)
            g0 = pltpu.async_copy(y_hbm.at[idx0_v], buf0, sem)
            g1 = pltpu.async_copy(y_hbm.at[idx1_v], buf1, sem)
            g0.wait()
            g1.wait()

            def row_body(r, _):
                a = w0_v[r]
                b = w1_v[r]

                def col_body(j, _):
                    sl = pl.ds(pl.multiple_of(j * 16, 16), 16)
                    outb[r, sl] = a * buf0[r, sl] + b * buf1[r, sl]
                    return 0

                return lax.fori_loop(0, D // 16, col_body, 0)

            lax.fori_loop(0, _CTPW, row_body, 0)
            pltpu.sync_copy(outb, out_hbm.at[pl.ds(base, _CTPW)])

    return dispatch, combine


def kernel(hidden_states, W1, b1, W2, b2, router_W, router_b, expert_bias):
    bsz, seq, dim = hidden_states.shape
    x2d = hidden_states.reshape(T, D)
    d0, d1, w0, w1, be = _route(x2d, router_W, router_b, expert_bias)
    d0 = d0.reshape(T)
    d1 = d1.reshape(T)
    be = be.reshape(32)[:NB]
    dispatch, combine = _sc_kernels()
    x_sorted = dispatch(x2d, d0, d1)
    y_sorted = _ffn(x_sorted, be, W1, b1, W2, b2)
    out = combine(y_sorted, d0, d1, w0, w1)
    return out.reshape(bsz, seq, dim)


# speedup vs baseline: 4.2861x; 1.0564x over previous
import functools

import jax
import jax.numpy as jnp
from jax import lax
from jax.experimental import pallas as pl
from jax.experimental.pallas import tpu as pltpu
from jax.experimental.pallas import tpu_sc as plsc

E = 8
TOP_K = 2
D = 768
F = 3072
T = 2048
BLK = 256
NB = T * TOP_K // BLK + E
P = NB * BLK
CH = 128
NCH = T // CH

_PREC = lax.Precision.HIGHEST


def _router_kernel(x_ref, rw_ref, rb_ref, eb_ref,
                   d0_ref, d1_ref, w0_ref, w1_ref, be_ref, exc_ref):
    x = x_ref[...]
    logits = jnp.dot(x, rw_ref[...], preferred_element_type=jnp.float32,
                     precision=lax.Precision.DEFAULT)
    logits = logits + rb_ref[...] + eb_ref[...]
    m = jnp.max(logits, axis=1, keepdims=True)
    ex = jnp.exp(logits - m)
    probs = ex / jnp.sum(ex, axis=1, keepdims=True)

    ei = lax.broadcasted_iota(jnp.int32, (T, E), 1)
    m1 = jnp.max(probs, axis=1, keepdims=True)
    i1 = jnp.min(jnp.where(probs == m1, ei, E), axis=1, keepdims=True)
    pmask = jnp.where(ei == i1, -1.0, probs)
    m2 = jnp.max(pmask, axis=1, keepdims=True)
    i2 = jnp.min(jnp.where(pmask == m2, ei, E), axis=1, keepdims=True)
    s = m1 + m2 + 1e-9
    w0_ref[...] = jnp.broadcast_to(m1 / s, (T, 16))
    w1_ref[...] = jnp.broadcast_to(m2 / s, (T, 16))

    oh1 = (ei == i1).astype(jnp.float32)
    oh2 = (ei == i2).astype(jnp.float32)
    occ = oh1 + oh2

    li = lax.broadcasted_iota(jnp.int32, (CH, CH), 0)
    lj = lax.broadcasted_iota(jnp.int32, (CH, CH), 1)
    lex = (lj < li).astype(jnp.float32)
    ones_row = jnp.ones((1, CH), jnp.float32)
    off = jnp.zeros((1, E), jnp.float32)
    for c in range(NCH):
        oc = occ[c * CH:(c + 1) * CH, :]
        exc_ref[c * CH:(c + 1) * CH, :] = off + jnp.dot(
            lex, oc, preferred_element_type=jnp.float32, precision=_PREC)
        off = off + jnp.dot(ones_row, oc,
                            preferred_element_type=jnp.float32,
                            precision=_PREC)
    counts = off

    pc = jnp.ceil(counts * (1.0 / BLK)) * BLK
    gi = lax.broadcasted_iota(jnp.int32, (E, E), 0)
    gj = lax.broadcasted_iota(jnp.int32, (E, E), 1)
    gmat = (gi < gj).astype(jnp.float32)
    gs = jnp.dot(pc, gmat, preferred_element_type=jnp.float32,
                 precision=_PREC)

    exc = exc_ref[...]
    slot = gs + exc
    d0_ref[...] = jnp.sum(oh1 * slot, axis=1, keepdims=True).astype(jnp.int32)
    d1_ref[...] = jnp.sum(oh2 * slot, axis=1, keepdims=True).astype(jnp.int32)

    brow = lax.broadcasted_iota(jnp.int32, (32, E), 0).astype(jnp.float32) * BLK
    ecol = lax.broadcasted_iota(jnp.int32, (32, E), 1)
    act = jnp.logical_and(brow >= gs, brow < gs + pc)
    be = jnp.sum(jnp.where(act, ecol + 1, 0), axis=1, keepdims=True) - 1
    be_ref[...] = be


def _route(x2d, router_W, router_b, expert_bias):
    out_shapes = (
        jax.ShapeDtypeStruct((T, 1), jnp.int32),
        jax.ShapeDtypeStruct((T, 1), jnp.int32),
        jax.ShapeDtypeStruct((T, 16), jnp.float32),
        jax.ShapeDtypeStruct((T, 16), jnp.float32),
        jax.ShapeDtypeStruct((32, 1), jnp.int32),
    )
    return pl.pallas_call(
        _router_kernel,
        out_shape=out_shapes,
        scratch_shapes=[pltpu.VMEM((T, E), jnp.float32)],
    )(x2d, router_W, router_b.reshape(1, E), expert_bias.reshape(1, E))


def _ffn_kernel(be_sref, x_ref, w1_ref, b1_ref, w2_ref, b2_ref, y_ref):
    b = pl.program_id(0)

    @pl.when(be_sref[b] >= 0)
    def _active():
        h = jnp.dot(x_ref[...], w1_ref[0], preferred_element_type=jnp.float32,
                    precision=lax.Precision.DEFAULT) + b1_ref[0]
        h = h * 0.5 * (1.0 + lax.erf(h * (2.0 ** -0.5)))
        y_ref[...] = jnp.dot(h, w2_ref[0], preferred_element_type=jnp.float32,
                             precision=lax.Precision.DEFAULT) + b2_ref[0]

    @pl.when(be_sref[b] < 0)
    def _inactive():
        y_ref[...] = jnp.zeros_like(y_ref)


def _ffn(x_sorted, block_expert, W1, b1, W2, b2):
    def wmap(b, be):
        return (jnp.where(be[b] < 0, E - 1, be[b]), 0, 0)

    grid_spec = pltpu.PrefetchScalarGridSpec(
        num_scalar_prefetch=1,
        grid=(NB,),
        in_specs=[
            pl.BlockSpec((BLK, D), lambda b, be: (b, 0)),
            pl.BlockSpec((1, D, F), wmap),
            pl.BlockSpec((1, 1, F), wmap),
            pl.BlockSpec((1, F, D), wmap),
            pl.BlockSpec((1, 1, D), wmap),
        ],
        out_specs=pl.BlockSpec((BLK, D), lambda b, be: (b, 0)),
    )
    return pl.pallas_call(
        _ffn_kernel,
        grid_spec=grid_spec,
        out_shape=jax.ShapeDtypeStruct((P, D), jnp.float32),
    )(block_expert, x_sorted, W1, b1.reshape(E, 1, F), W2,
      b2.reshape(E, 1, D))


_NC = 2
_NS = 16
_NW = _NC * _NS
_TPW = T // _NW


_CTPW = 32


@functools.cache
def _sc_kernels():
    mesh = plsc.VectorSubcoreMesh(core_axis_name="c", subcore_axis_name="s")

    @functools.partial(
        pl.kernel,
        mesh=mesh,
        out_type=jax.ShapeDtypeStruct((P, D), jnp.float32),
        scratch_types=[
            pltpu.VMEM((_TPW,), jnp.int32),
            pltpu.VMEM((_TPW,), jnp.int32),
            pltpu.VMEM((_TPW, D), jnp.float32),
            pltpu.SemaphoreType.DMA,
        ],
    )
    def dispatch(x_hbm, d0_hbm, d1_hbm, xs_hbm, idx0_v, idx1_v, rows_v, sem):
        wid = lax.axis_index("s") * _NC + lax.axis_index("c")
        base = wid * _TPW
        pltpu.sync_copy(d0_hbm.at[pl.ds(base, _TPW)], idx0_v)
        pltpu.sync_copy(d1_hbm.at[pl.ds(base, _TPW)], idx1_v)
        pltpu.sync_copy(x_hbm.at[pl.ds(base, _TPW)], rows_v)
        pltpu.async_copy(rows_v, xs_hbm.at[idx0_v], sem).wait()
        pltpu.async_copy(rows_v, xs_hbm.at[idx1_v], sem).wait()

    @functools.partial(
        pl.kernel,
        mesh=mesh,
        out_type=jax.ShapeDtypeStruct((T, D), jnp.float32),
        scratch_types=[
            pltpu.VMEM((_CTPW,), jnp.int32),
            pltpu.VMEM((_CTPW,), jnp.int32),
            pltpu.VMEM((_CTPW, 16), jnp.float32),
            pltpu.VMEM((_CTPW, 16), jnp.float32),
            pltpu.VMEM((_CTPW, D), jnp.float32),
            pltpu.VMEM((_CTPW, D), jnp.float32),
            pltpu.VMEM((_CTPW, D), jnp.float32),
            pltpu.SemaphoreType.DMA,
        ],
    )
    def combine(y_hbm, d0_hbm, d1_hbm, w0_hbm, w1_hbm, out_hbm,
                idx0_v, idx1_v, w0_v, w1_v, buf0, buf1, outb, sem):
        wid = lax.axis_index("s") * _NC + lax.axis_index("c")
        for c in range(_TPW // _CTPW):
            base = wid * _TPW + c * _CTPW
            pltpu.sync_copy(d0_hbm.at[pl.ds(base, _CTPW)], idx0_v)
            pltpu.sync_copy(d1_hbm.at[pl.ds(base, _CTPW)], idx1_v)
            pltpu.sync_copy(w0_hbm.at[pl.ds(base, _CTPW)], w0_v)
            pltpu.sync_copy(w1_hbm.at[pl.ds(base, _CTPW)], w1_v)
            g0 = pltpu.async_copy(y_hbm.at[idx0_v], buf0, sem)
            g1 = pltpu.async_copy(y_hbm.at[idx1_v], buf1, sem)
            g0.wait()
            g1.wait()

            def row_body(r, _):
                a = w0_v[r]
                b = w1_v[r]
                for j in range(D // 16):
                    sl = pl.ds(j * 16, 16)
                    outb[r, sl] = a * buf0[r, sl] + b * buf1[r, sl]
                return 0

            lax.fori_loop(0, _CTPW, row_body, 0)
            pltpu.sync_copy(outb, out_hbm.at[pl.ds(base, _CTPW)])

    return dispatch, combine


def kernel(hidden_states, W1, b1, W2, b2, router_W, router_b, expert_bias):
    bsz, seq, dim = hidden_states.shape
    x2d = hidden_states.reshape(T, D)
    d0, d1, w0, w1, be = _route(x2d, router_W, router_b, expert_bias)
    d0 = d0.reshape(T)
    d1 = d1.reshape(T)
    be = be.reshape(32)[:NB]
    dispatch, combine = _sc_kernels()
    x_sorted = dispatch(x2d, d0, d1)
    y_sorted = _ffn(x_sorted, be, W1, b1, W2, b2)
    out = combine(y_sorted, d0, d1, w0, w1)
    return out.reshape(bsz, seq, dim)


# speedup vs baseline: 4.3442x; 1.0135x over previous
import functools

import jax
import jax.numpy as jnp
from jax import lax
from jax.experimental import pallas as pl
from jax.experimental.pallas import tpu as pltpu
from jax.experimental.pallas import tpu_sc as plsc

E = 8
TOP_K = 2
D = 768
F = 3072
T = 2048
BLK = 256
NB = T * TOP_K // BLK + E
P = NB * BLK
CH = 128
NCH = T // CH

_PREC = lax.Precision.HIGHEST


def _router_kernel(x_ref, rw_ref, rb_ref, eb_ref,
                   d0_ref, d1_ref, w0_ref, w1_ref, be_ref, exc_ref):
    x = x_ref[...]
    logits = jnp.dot(x, rw_ref[...], preferred_element_type=jnp.float32,
                     precision=lax.Precision.DEFAULT)
    logits = logits + rb_ref[...] + eb_ref[...]
    m = jnp.max(logits, axis=1, keepdims=True)
    ex = jnp.exp(logits - m)
    probs = ex / jnp.sum(ex, axis=1, keepdims=True)

    ei = lax.broadcasted_iota(jnp.int32, (T, E), 1)
    m1 = jnp.max(probs, axis=1, keepdims=True)
    i1 = jnp.min(jnp.where(probs == m1, ei, E), axis=1, keepdims=True)
    pmask = jnp.where(ei == i1, -1.0, probs)
    m2 = jnp.max(pmask, axis=1, keepdims=True)
    i2 = jnp.min(jnp.where(pmask == m2, ei, E), axis=1, keepdims=True)
    s = m1 + m2 + 1e-9
    w0_ref[...] = jnp.broadcast_to(m1 / s, (T, 16))
    w1_ref[...] = jnp.broadcast_to(m2 / s, (T, 16))

    oh1 = (ei == i1).astype(jnp.float32)
    oh2 = (ei == i2).astype(jnp.float32)
    occ = oh1 + oh2

    li = lax.broadcasted_iota(jnp.int32, (CH, CH), 0)
    lj = lax.broadcasted_iota(jnp.int32, (CH, CH), 1)
    lex = (lj < li).astype(jnp.float32)
    ones_row = jnp.ones((1, CH), jnp.float32)
    off = jnp.zeros((1, E), jnp.float32)
    for c in range(NCH):
        oc = occ[c * CH:(c + 1) * CH, :]
        exc_ref[c * CH:(c + 1) * CH, :] = off + jnp.dot(
            lex, oc, preferred_element_type=jnp.float32,
            precision=lax.Precision.DEFAULT)
        off = off + jnp.dot(ones_row, oc,
                            preferred_element_type=jnp.float32,
                            precision=lax.Precision.DEFAULT)
    counts = off

    pc = jnp.ceil(counts * (1.0 / BLK)) * BLK
    gi = lax.broadcasted_iota(jnp.int32, (E, E), 0)
    gj = lax.broadcasted_iota(jnp.int32, (E, E), 1)
    gmat = (gi < gj).astype(jnp.float32)
    gs = jnp.dot(pc, gmat, preferred_element_type=jnp.float32,
                 precision=lax.Precision.DEFAULT)

    exc = exc_ref[...]
    slot = gs + exc
    d0_ref[...] = jnp.sum(oh1 * slot, axis=1, keepdims=True).astype(jnp.int32)
    d1_ref[...] = jnp.sum(oh2 * slot, axis=1, keepdims=True).astype(jnp.int32)

    brow = lax.broadcasted_iota(jnp.int32, (32, E), 0).astype(jnp.float32) * BLK
    ecol = lax.broadcasted_iota(jnp.int32, (32, E), 1)
    act = jnp.logical_and(brow >= gs, brow < gs + pc)
    be = jnp.sum(jnp.where(act, ecol + 1, 0), axis=1, keepdims=True) - 1
    be_ref[...] = be


def _route(x2d, router_W, router_b, expert_bias):
    out_shapes = (
        jax.ShapeDtypeStruct((T, 1), jnp.int32),
        jax.ShapeDtypeStruct((T, 1), jnp.int32),
        jax.ShapeDtypeStruct((T, 16), jnp.float32),
        jax.ShapeDtypeStruct((T, 16), jnp.float32),
        jax.ShapeDtypeStruct((32, 1), jnp.int32),
    )
    return pl.pallas_call(
        _router_kernel,
        out_shape=out_shapes,
        scratch_shapes=[pltpu.VMEM((T, E), jnp.float32)],
    )(x2d, router_W, router_b.reshape(1, E), expert_bias.reshape(1, E))


def _ffn_kernel(be_sref, x_ref, w1_ref, b1_ref, w2_ref, b2_ref, y_ref):
    b = pl.program_id(0)

    @pl.when(be_sref[b] >= 0)
    def _active():
        h = jnp.dot(x_ref[...], w1_ref[0], preferred_element_type=jnp.float32,
                    precision=lax.Precision.DEFAULT) + b1_ref[0]
        h = h * 0.5 * (1.0 + lax.erf(h * (2.0 ** -0.5)))
        y_ref[...] = jnp.dot(h, w2_ref[0], preferred_element_type=jnp.float32,
                             precision=lax.Precision.DEFAULT) + b2_ref[0]

    @pl.when(be_sref[b] < 0)
    def _inactive():
        y_ref[...] = jnp.zeros_like(y_ref)


def _ffn(x_sorted, block_expert, W1, b1, W2, b2):
    def wmap(b, be):
        return (jnp.where(be[b] < 0, E - 1, be[b]), 0, 0)

    grid_spec = pltpu.PrefetchScalarGridSpec(
        num_scalar_prefetch=1,
        grid=(NB,),
        in_specs=[
            pl.BlockSpec((BLK, D), lambda b, be: (b, 0)),
            pl.BlockSpec((1, D, F), wmap),
            pl.BlockSpec((1, 1, F), wmap),
            pl.BlockSpec((1, F, D), wmap),
            pl.BlockSpec((1, 1, D), wmap),
        ],
        out_specs=pl.BlockSpec((BLK, D), lambda b, be: (b, 0)),
    )
    return pl.pallas_call(
        _ffn_kernel,
        grid_spec=grid_spec,
        out_shape=jax.ShapeDtypeStruct((P, D), jnp.float32),
    )(block_expert, x_sorted, W1, b1.reshape(E, 1, F), W2,
      b2.reshape(E, 1, D))


_NC = 2
_NS = 16
_NW = _NC * _NS
_TPW = T // _NW


_CTPW = 32


@functools.cache
def _sc_kernels():
    mesh = plsc.VectorSubcoreMesh(core_axis_name="c", subcore_axis_name="s")

    @functools.partial(
        pl.kernel,
        mesh=mesh,
        out_type=jax.ShapeDtypeStruct((P, D), jnp.float32),
        scratch_types=[
            pltpu.VMEM((_TPW,), jnp.int32),
            pltpu.VMEM((_TPW,), jnp.int32),
            pltpu.VMEM((_TPW, D), jnp.float32),
            pltpu.SemaphoreType.DMA,
        ],
    )
    def dispatch(x_hbm, d0_hbm, d1_hbm, xs_hbm, idx0_v, idx1_v, rows_v, sem):
        wid = lax.axis_index("s") * _NC + lax.axis_index("c")
        base = wid * _TPW
        pltpu.sync_copy(d0_hbm.at[pl.ds(base, _TPW)], idx0_v)
        pltpu.sync_copy(d1_hbm.at[pl.ds(base, _TPW)], idx1_v)
        pltpu.sync_copy(x_hbm.at[pl.ds(base, _TPW)], rows_v)
        c0 = pltpu.async_copy(rows_v, xs_hbm.at[idx0_v], sem)
        c1 = pltpu.async_copy(rows_v, xs_hbm.at[idx1_v], sem)
        c0.wait()
        c1.wait()

    @functools.partial(
        pl.kernel,
        mesh=mesh,
        out_type=jax.ShapeDtypeStruct((T, D), jnp.float32),
        scratch_types=[
            pltpu.VMEM((_CTPW,), jnp.int32),
            pltpu.VMEM((_CTPW,), jnp.int32),
            pltpu.VMEM((_CTPW, 16), jnp.float32),
            pltpu.VMEM((_CTPW, 16), jnp.float32),
            pltpu.VMEM((_CTPW, D), jnp.float32),
            pltpu.VMEM((_CTPW, D), jnp.float32),
            pltpu.VMEM((_CTPW, D), jnp.float32),
            pltpu.SemaphoreType.DMA,
        ],
    )
    def combine(y_hbm, d0_hbm, d1_hbm, w0_hbm, w1_hbm, out_hbm,
                idx0_v, idx1_v, w0_v, w1_v, buf0, buf1, outb, sem):
        wid = lax.axis_index("s") * _NC + lax.axis_index("c")
        for c in range(_TPW // _CTPW):
            base = wid * _TPW + c * _CTPW
            pltpu.sync_copy(d0_hbm.at[pl.ds(base, _CTPW)], idx0_v)
            pltpu.sync_copy(d1_hbm.at[pl.ds(base, _CTPW)], idx1_v)
            pltpu.sync_copy(w0_hbm.at[pl.ds(base, _CTPW)], w0_v)
            pltpu.sync_copy(w1_hbm.at[pl.ds(base, _CTPW)], w1_v)
            g0 = pltpu.async_copy(y_hbm.at[idx0_v], buf0, sem)
            g1 = pltpu.async_copy(y_hbm.at[idx1_v], buf1, sem)
            g0.wait()
            g1.wait()

            def row_body(r, _):
                a = w0_v[r]
                b = w1_v[r]
                for j in range(D // 16):
                    sl = pl.ds(j * 16, 16)
                    outb[r, sl] = a * buf0[r, sl] + b * buf1[r, sl]
                return 0

            lax.fori_loop(0, _CTPW, row_body, 0)
            pltpu.sync_copy(outb, out_hbm.at[pl.ds(base, _CTPW)])

    return dispatch, combine


def kernel(hidden_states, W1, b1, W2, b2, router_W, router_b, expert_bias):
    bsz, seq, dim = hidden_states.shape
    x2d = hidden_states.reshape(T, D)
    d0, d1, w0, w1, be = _route(x2d, router_W, router_b, expert_bias)
    d0 = d0.reshape(T)
    d1 = d1.reshape(T)
    be = be.reshape(32)[:NB]
    dispatch, combine = _sc_kernels()
    x_sorted = dispatch(x2d, d0, d1)
    y_sorted = _ffn(x_sorted, be, W1, b1, W2, b2)
    out = combine(y_sorted, d0, d1, w0, w1)
    return out.reshape(bsz, seq, dim)
